# baseline (device time: 25138 ns/iter reference)
import jax
import jax.numpy as jnp
from jax import lax
from jax.experimental import pallas as pl
from jax.experimental.pallas import tpu as pltpu


def kernel(x, router, W1, W2):
    t_loc, d = x.shape
    e_loc, _, f = W1.shape
    e_tot = 2 * e_loc

    def body(x_ref, r_ref, w1_ref, w2_ref, out_ref,
             w1v, w2v, xsend, xpeer, rpeer, wsend, wrecv, psend, pcomb,
             local_sems, send_sems, recv_sems):
        my_x = lax.axis_index("x")
        my_y = lax.axis_index("y")
        my_z = lax.axis_index("z")
        peer = (1 - my_x, my_y, my_z)

        cp1 = pltpu.make_async_copy(w1_ref, w1v, local_sems.at[0])
        cp2 = pltpu.make_async_copy(w2_ref, w2v, local_sems.at[1])
        cp1.start()
        cp2.start()

        barrier = pltpu.get_barrier_semaphore()
        pl.semaphore_signal(barrier, inc=1, device_id=peer,
                            device_id_type=pl.DeviceIdType.MESH)
        pl.semaphore_wait(barrier, 1)

        xsend[:, :] = x_ref[:, :].astype(jnp.bfloat16)
        rdma_r = pltpu.make_async_remote_copy(
            src_ref=r_ref, dst_ref=rpeer,
            send_sem=send_sems.at[0], recv_sem=recv_sems.at[0],
            device_id=peer, device_id_type=pl.DeviceIdType.MESH)
        rdma_x = pltpu.make_async_remote_copy(
            src_ref=xsend, dst_ref=xpeer,
            send_sem=send_sems.at[1], recv_sem=recv_sems.at[1],
            device_id=peer, device_id_type=pl.DeviceIdType.MESH)
        rdma_r.start()
        rdma_x.start()
        rdma_r.wait()

        rcat = jnp.concatenate([r_ref[:, :], rpeer[:, :]], axis=1)
        g = jnp.dot(x_ref[:, :], rcat,
                    preferred_element_type=jnp.float32)
        col = lax.broadcasted_iota(jnp.int32, g.shape, 1)
        a1 = jnp.argmax(g, axis=1)[:, None]
        oh1 = (col == a1).astype(jnp.float32)
        m1 = jnp.max(g, axis=1, keepdims=True)
        gmask = jnp.where(col == a1, -jnp.inf, g)
        a2 = jnp.argmax(gmask, axis=1)[:, None]
        oh2 = (col == a2).astype(jnp.float32)
        m2 = jnp.max(gmask, axis=1, keepdims=True)
        b = jnp.exp(m2 - m1)
        wgt = (oh1 + oh2 * b) / (1.0 + b)
        wsend[:, :] = wgt

        rdma_w = pltpu.make_async_remote_copy(
            src_ref=wsend, dst_ref=wrecv,
            send_sem=send_sems.at[2], recv_sem=recv_sems.at[2],
            device_id=peer, device_id_type=pl.DeviceIdType.MESH)
        rdma_w.start()

        cp1.wait()
        cp2.wait()
        xl = x_ref[:, :].astype(jnp.bfloat16)
        w1b = [w1v[j].astype(jnp.bfloat16) for j in range(e_loc)]
        w2b = [w2v[j].astype(jnp.bfloat16) for j in range(e_loc)]
        acc = jnp.zeros((t_loc, d), jnp.float32)
        for j in range(e_loc):
            h = jnp.maximum(
                jnp.dot(xl, w1b[j], preferred_element_type=jnp.float32),
                0.0).astype(jnp.bfloat16)
            acc = acc + jnp.dot(
                h, w2b[j], preferred_element_type=jnp.float32
            ) * wgt[:, j:j + 1]

        rdma_x.wait()
        rdma_w.wait()
        wp = wrecv[:, e_loc:e_tot]
        pacc = jnp.zeros((t_loc, d), jnp.float32)
        for j in range(e_loc):
            h = jnp.maximum(
                jnp.dot(xpeer[:, :], w1b[j],
                        preferred_element_type=jnp.float32),
                0.0).astype(jnp.bfloat16)
            pacc = pacc + jnp.dot(
                h, w2b[j], preferred_element_type=jnp.float32
            ) * wp[:, j:j + 1]
        psend[:, :] = pacc.astype(jnp.bfloat16)

        rdma_p = pltpu.make_async_remote_copy(
            src_ref=psend, dst_ref=pcomb,
            send_sem=send_sems.at[3], recv_sem=recv_sems.at[3],
            device_id=peer, device_id_type=pl.DeviceIdType.MESH)
        rdma_p.start()
        rdma_p.wait()

        out_ref[:, :] = acc + pcomb[:, :].astype(jnp.float32)

    return pl.pallas_call(
        body,
        out_shape=jax.ShapeDtypeStruct((t_loc, d), jnp.float32),
        in_specs=[
            pl.BlockSpec(memory_space=pltpu.VMEM),
            pl.BlockSpec(memory_space=pltpu.VMEM),
            pl.BlockSpec(memory_space=pl.ANY),
            pl.BlockSpec(memory_space=pl.ANY),
        ],
        out_specs=pl.BlockSpec(memory_space=pltpu.VMEM),
        scratch_shapes=[
            pltpu.VMEM((e_loc, d, f), jnp.float32),
            pltpu.VMEM((e_loc, f, d), jnp.float32),
            pltpu.VMEM((t_loc, d), jnp.bfloat16),
            pltpu.VMEM((t_loc, d), jnp.bfloat16),
            pltpu.VMEM((d, e_loc), jnp.float32),
            pltpu.VMEM((t_loc, e_tot), jnp.float32),
            pltpu.VMEM((t_loc, e_tot), jnp.float32),
            pltpu.VMEM((t_loc, d), jnp.bfloat16),
            pltpu.VMEM((t_loc, d), jnp.bfloat16),
            pltpu.SemaphoreType.DMA((2,)),
            pltpu.SemaphoreType.DMA((4,)),
            pltpu.SemaphoreType.DMA((4,)),
        ],
        compiler_params=pltpu.CompilerParams(collective_id=0),
    )(x, router, W1, W2)


# device time: 23399 ns/iter; 1.0743x vs baseline; 1.0743x over previous
import jax
import jax.numpy as jnp
from jax import lax
from jax.experimental import pallas as pl
from jax.experimental.pallas import tpu as pltpu


def kernel(x, router, W1, W2):
    t_loc, d = x.shape
    e_loc, _, f = W1.shape
    e_tot = 2 * e_loc

    def body(x_ref, r_ref, w1_ref, w2_ref, out_ref,
             w1v, w2v, xsend, xpeer, rpeer, wsend, wrecv, psend, pcomb,
             local_sems, send_sems, recv_sems):
        my_x = lax.axis_index("x")
        my_y = lax.axis_index("y")
        my_z = lax.axis_index("z")
        peer = (1 - my_x, my_y, my_z)

        cp1 = pltpu.make_async_copy(w1_ref, w1v, local_sems.at[0])
        cp2 = pltpu.make_async_copy(w2_ref, w2v, local_sems.at[1])
        cp1.start()
        cp2.start()

        barrier = pltpu.get_barrier_semaphore()
        pl.semaphore_signal(barrier, inc=1, device_id=peer,
                            device_id_type=pl.DeviceIdType.MESH)
        pl.semaphore_wait(barrier, 1)

        xsend[:, :] = x_ref[:, :].astype(jnp.bfloat16)
        rdma_r = pltpu.make_async_remote_copy(
            src_ref=r_ref, dst_ref=rpeer,
            send_sem=send_sems.at[0], recv_sem=recv_sems.at[0],
            device_id=peer, device_id_type=pl.DeviceIdType.MESH)
        rdma_x = pltpu.make_async_remote_copy(
            src_ref=xsend, dst_ref=xpeer,
            send_sem=send_sems.at[1], recv_sem=recv_sems.at[1],
            device_id=peer, device_id_type=pl.DeviceIdType.MESH)
        rdma_r.start()
        rdma_x.start()
        rdma_r.wait()

        rcat = jnp.concatenate([r_ref[:, :], rpeer[:, :]], axis=1)
        g = jnp.dot(x_ref[:, :], rcat,
                    preferred_element_type=jnp.float32)
        col = lax.broadcasted_iota(jnp.int32, g.shape, 1)
        a1 = jnp.argmax(g, axis=1)[:, None]
        oh1 = (col == a1).astype(jnp.float32)
        m1 = jnp.max(g, axis=1, keepdims=True)
        gmask = jnp.where(col == a1, -jnp.inf, g)
        a2 = jnp.argmax(gmask, axis=1)[:, None]
        oh2 = (col == a2).astype(jnp.float32)
        m2 = jnp.max(gmask, axis=1, keepdims=True)
        b = jnp.exp(m2 - m1)
        wgt = (oh1 + oh2 * b) / (1.0 + b)
        wsend[:, :] = wgt

        rdma_w = pltpu.make_async_remote_copy(
            src_ref=wsend, dst_ref=wrecv,
            send_sem=send_sems.at[2], recv_sem=recv_sems.at[2],
            device_id=peer, device_id_type=pl.DeviceIdType.MESH)
        rdma_w.start()

        cp1.wait()
        cp2.wait()
        xl = x_ref[:, :].astype(jnp.bfloat16)
        acc = jnp.zeros((t_loc, d), jnp.float32)
        for j in range(e_loc):
            h = jnp.maximum(
                jnp.dot(xl, w1v[j], preferred_element_type=jnp.float32),
                0.0).astype(jnp.bfloat16)
            acc = acc + jnp.dot(
                h, w2v[j], preferred_element_type=jnp.float32
            ) * wgt[:, j:j + 1]

        rdma_x.wait()
        rdma_w.wait()
        wp = wrecv[:, e_loc:e_tot]
        pacc = jnp.zeros((t_loc, d), jnp.float32)
        for j in range(e_loc):
            h = jnp.maximum(
                jnp.dot(xpeer[:, :], w1v[j],
                        preferred_element_type=jnp.float32),
                0.0).astype(jnp.bfloat16)
            pacc = pacc + jnp.dot(
                h, w2v[j], preferred_element_type=jnp.float32
            ) * wp[:, j:j + 1]
        psend[:, :] = pacc.astype(jnp.bfloat16)

        rdma_p = pltpu.make_async_remote_copy(
            src_ref=psend, dst_ref=pcomb,
            send_sem=send_sems.at[3], recv_sem=recv_sems.at[3],
            device_id=peer, device_id_type=pl.DeviceIdType.MESH)
        rdma_p.start()
        rdma_p.wait()

        out_ref[:, :] = acc + pcomb[:, :].astype(jnp.float32)

    w1b = W1.astype(jnp.bfloat16)
    w2b = W2.astype(jnp.bfloat16)

    return pl.pallas_call(
        body,
        out_shape=jax.ShapeDtypeStruct((t_loc, d), jnp.float32),
        in_specs=[
            pl.BlockSpec(memory_space=pltpu.VMEM),
            pl.BlockSpec(memory_space=pltpu.VMEM),
            pl.BlockSpec(memory_space=pl.ANY),
            pl.BlockSpec(memory_space=pl.ANY),
        ],
        out_specs=pl.BlockSpec(memory_space=pltpu.VMEM),
        scratch_shapes=[
            pltpu.VMEM((e_loc, d, f), jnp.bfloat16),
            pltpu.VMEM((e_loc, f, d), jnp.bfloat16),
            pltpu.VMEM((t_loc, d), jnp.bfloat16),
            pltpu.VMEM((t_loc, d), jnp.bfloat16),
            pltpu.VMEM((d, e_loc), jnp.float32),
            pltpu.VMEM((t_loc, e_tot), jnp.float32),
            pltpu.VMEM((t_loc, e_tot), jnp.float32),
            pltpu.VMEM((t_loc, d), jnp.bfloat16),
            pltpu.VMEM((t_loc, d), jnp.bfloat16),
            pltpu.SemaphoreType.DMA((2,)),
            pltpu.SemaphoreType.DMA((4,)),
            pltpu.SemaphoreType.DMA((4,)),
        ],
        compiler_params=pltpu.CompilerParams(collective_id=0),
    )(x, router, w1b, w2b)


# device time: 21481 ns/iter; 1.1702x vs baseline; 1.0893x over previous
import jax
import jax.numpy as jnp
from jax import lax
from jax.experimental import pallas as pl
from jax.experimental.pallas import tpu as pltpu


def kernel(x, router, W1, W2):
    t_loc, d = x.shape
    e_loc, _, f = W1.shape
    e_tot = 2 * e_loc
    half = t_loc // 2

    def body(x_ref, r_ref, w1_ref, w2_ref, out_ref,
             w1v, w2v, xsend, xpeer, rpeer, wsend, wrecv, psend, pcomb,
             local_sems, send_sems, recv_sems):
        my_x = lax.axis_index("x")
        my_y = lax.axis_index("y")
        my_z = lax.axis_index("z")
        peer = (1 - my_x, my_y, my_z)

        cp1 = pltpu.make_async_copy(w1_ref, w1v, local_sems.at[0])
        cp2 = pltpu.make_async_copy(w2_ref, w2v, local_sems.at[1])
        cp1.start()
        cp2.start()

        barrier = pltpu.get_barrier_semaphore()
        pl.semaphore_signal(barrier, inc=1, device_id=peer,
                            device_id_type=pl.DeviceIdType.MESH)
        pl.semaphore_wait(barrier, 1)

        def rdma(src, dst, i):
            return pltpu.make_async_remote_copy(
                src_ref=src, dst_ref=dst,
                send_sem=send_sems.at[i], recv_sem=recv_sems.at[i],
                device_id=peer, device_id_type=pl.DeviceIdType.MESH)

        xsend[:, :] = x_ref[:, :].astype(jnp.bfloat16)
        rdma_r = rdma(r_ref, rpeer, 0)
        rdma_x1 = rdma(xsend.at[pl.ds(0, half)], xpeer.at[pl.ds(0, half)], 1)
        rdma_r.start()
        rdma_x1.start()
        rdma_r.wait()

        rcat = jnp.concatenate([r_ref[:, :], rpeer[:, :]], axis=1)
        g = jnp.dot(x_ref[:, :], rcat,
                    preferred_element_type=jnp.float32)
        col = lax.broadcasted_iota(jnp.int32, g.shape, 1)
        a1 = jnp.argmax(g, axis=1)[:, None]
        oh1 = (col == a1).astype(jnp.float32)
        m1 = jnp.max(g, axis=1, keepdims=True)
        gmask = jnp.where(col == a1, -jnp.inf, g)
        a2 = jnp.argmax(gmask, axis=1)[:, None]
        oh2 = (col == a2).astype(jnp.float32)
        m2 = jnp.max(gmask, axis=1, keepdims=True)
        b = jnp.exp(m2 - m1)
        wgt = (oh1 + oh2 * b) / (1.0 + b)
        wsend[:, :] = wgt

        rdma_w = rdma(wsend, wrecv, 2)
        rdma_x2 = rdma(xsend.at[pl.ds(half, half)],
                       xpeer.at[pl.ds(half, half)], 3)
        rdma_w.start()
        rdma_x2.start()

        cp1.wait()
        cp2.wait()
        xl = x_ref[:, :].astype(jnp.bfloat16)
        acc = jnp.zeros((t_loc, d), jnp.float32)
        for j in range(e_loc):
            h = jnp.maximum(
                jnp.dot(xl, w1v[j], preferred_element_type=jnp.float32),
                0.0).astype(jnp.bfloat16)
            acc = acc + jnp.dot(
                h, w2v[j], preferred_element_type=jnp.float32
            ) * wgt[:, j:j + 1]

        rdma_x1.wait()
        rdma_w.wait()
        wp = wrecv[:, e_loc:e_tot]

        def peer_chunk(lo):
            pacc = jnp.zeros((half, d), jnp.float32)
            for j in range(e_loc):
                h = jnp.maximum(
                    jnp.dot(xpeer[lo:lo + half, :], w1v[j],
                            preferred_element_type=jnp.float32),
                    0.0).astype(jnp.bfloat16)
                pacc = pacc + jnp.dot(
                    h, w2v[j], preferred_element_type=jnp.float32
                ) * wp[lo:lo + half, j:j + 1]
            return pacc.astype(jnp.bfloat16)

        psend[pl.ds(0, half), :] = peer_chunk(0)
        rdma_p1 = rdma(psend.at[pl.ds(0, half)], pcomb.at[pl.ds(0, half)], 4)
        rdma_p1.start()

        rdma_x2.wait()
        psend[pl.ds(half, half), :] = peer_chunk(half)
        rdma_p2 = rdma(psend.at[pl.ds(half, half)],
                       pcomb.at[pl.ds(half, half)], 5)
        rdma_p2.start()

        rdma_p1.wait()
        out_ref[pl.ds(0, half), :] = (
            acc[:half] + pcomb[0:half, :].astype(jnp.float32))
        rdma_p2.wait()
        out_ref[pl.ds(half, half), :] = (
            acc[half:] + pcomb[half:t_loc, :].astype(jnp.float32))

    w1b = W1.astype(jnp.bfloat16)
    w2b = W2.astype(jnp.bfloat16)

    return pl.pallas_call(
        body,
        out_shape=jax.ShapeDtypeStruct((t_loc, d), jnp.float32),
        in_specs=[
            pl.BlockSpec(memory_space=pltpu.VMEM),
            pl.BlockSpec(memory_space=pltpu.VMEM),
            pl.BlockSpec(memory_space=pl.ANY),
            pl.BlockSpec(memory_space=pl.ANY),
        ],
        out_specs=pl.BlockSpec(memory_space=pltpu.VMEM),
        scratch_shapes=[
            pltpu.VMEM((e_loc, d, f), jnp.bfloat16),
            pltpu.VMEM((e_loc, f, d), jnp.bfloat16),
            pltpu.VMEM((t_loc, d), jnp.bfloat16),
            pltpu.VMEM((t_loc, d), jnp.bfloat16),
            pltpu.VMEM((d, e_loc), jnp.float32),
            pltpu.VMEM((t_loc, e_tot), jnp.float32),
            pltpu.VMEM((t_loc, e_tot), jnp.float32),
            pltpu.VMEM((t_loc, d), jnp.bfloat16),
            pltpu.VMEM((t_loc, d), jnp.bfloat16),
            pltpu.SemaphoreType.DMA((2,)),
            pltpu.SemaphoreType.DMA((6,)),
            pltpu.SemaphoreType.DMA((6,)),
        ],
        compiler_params=pltpu.CompilerParams(collective_id=0),
    )(x, router, w1b, w2b)
